# Initial kernel scaffold; baseline (speedup 1.0000x reference)
import jax
import jax.numpy as jnp
from jax.experimental import pallas as pl


def kernel(x):
    raise NotImplementedError("write your pallas kernel here")


# speedup vs baseline: 37.6540x; 37.6540x over previous
import numpy as np
import jax
import jax.numpy as jnp
from jax.experimental import pallas as pl
from jax.experimental.pallas import tpu as pltpu

_NUMANGLE = 100
_NUMRHO = 100
_B, _C, _H, _W = 1, 128, 100, 100
_P = _H * _W


def _make_rho_table():
    irho = int(np.sqrt(_H * _H + _W * _W) + 1) / float(_NUMRHO - 1)
    itheta = np.pi / _NUMANGLE
    angles = np.arange(_NUMANGLE) * itheta
    tab_cos = (np.cos(angles) / irho).astype(np.float32)
    tab_sin = (np.sin(angles) / irho).astype(np.float32)
    ys, xs = np.meshgrid(np.arange(_H), np.arange(_W), indexing="ij")
    xx = (xs - _W // 2).astype(np.float32)
    yy = (ys - _H // 2).astype(np.float32)
    r = np.round(xx[None] * tab_cos[:, None, None] + yy[None] * tab_sin[:, None, None])
    r = r.astype(np.int32) + _NUMRHO // 2
    return r.reshape(_NUMANGLE, _P)


_RTAB = _make_rho_table()


def _dht_body(r_ref, x_ref, out_ref, v_ref):
    a = pl.program_id(0)

    @pl.when(a == 0)
    def _():
        v_ref[...] = jnp.maximum(x_ref[...], 0.0)

    r = r_ref[0, 0, :]
    iota = jax.lax.broadcasted_iota(jnp.int32, (_P, _NUMRHO), 1)
    onehot = jnp.where(r[:, None] == iota, 1.0, 0.0).astype(jnp.float32)
    out_ref[0] = jnp.dot(v_ref[...], onehot, preferred_element_type=jnp.float32)


def kernel(x):
    r3 = jnp.asarray(_RTAB)[:, None, :]
    v2 = x.reshape(_C, _P)
    out = pl.pallas_call(
        _dht_body,
        grid=(_NUMANGLE,),
        in_specs=[
            pl.BlockSpec((1, 1, _P), lambda a: (a, 0, 0)),
            pl.BlockSpec((_C, _P), lambda a: (0, 0)),
        ],
        out_specs=pl.BlockSpec((1, _C, _NUMRHO), lambda a: (a, 0, 0)),
        out_shape=jax.ShapeDtypeStruct((_NUMANGLE, _C, _NUMRHO), jnp.float32),
        scratch_shapes=[pltpu.VMEM((_C, _P), jnp.float32)],
    )(r3, v2)
    return out.transpose(1, 0, 2).reshape(_B, _C, _NUMANGLE, _NUMRHO)
